# baseline (speedup 1.0000x reference)
import functools

import jax
import jax.numpy as jnp
from jax import lax
from jax.experimental import pallas as pl
from jax.experimental.pallas import tpu as pltpu
from jax.experimental.pallas import tpu_sc as plsc

NC = 2
NS = 16
NW = NC * NS
L = 16
CH = 2048
U = 8


def _make_sc_call(n_cuts, n_seg):
    S = n_seg // NW
    SP = S + 8
    ACC = 16 * SP
    IPW = S + 32
    mesh = plsc.VectorSubcoreMesh(core_axis_name="c", subcore_axis_name="s")

    @functools.partial(
        pl.kernel,
        mesh=mesh,
        compiler_params=pltpu.CompilerParams(needs_layout_passes=False),
        out_type=jax.ShapeDtypeStruct((16 * n_seg,), jnp.float32),
        scratch_types=[
            pltpu.VMEM((IPW,), jnp.int32),
            pltpu.VMEM((16,), jnp.float32),
            pltpu.VMEM((17 * 16,), jnp.float32),
            pltpu.VMEM((CH,), jnp.float32),
            pltpu.VMEM((CH,), jnp.float32),
            pltpu.VMEM((ACC,), jnp.float32),
            pltpu.SemaphoreType.DMA,
            pltpu.SemaphoreType.DMA,
        ],
    )
    def sc_call(coords_hbm, ip_hbm, bins_hbm, out_hbm,
                ip_v, bins_v, bins_x, cbuf0, cbuf1, acc, sem0, sem1):
        cid = lax.axis_index("c")
        sid = lax.axis_index("s")
        w = sid * NC + cid
        pltpu.sync_copy(ip_hbm.at[pl.ds(w * S, IPW)], ip_v)
        pltpu.sync_copy(bins_hbm, bins_v)

        zero16 = jnp.zeros((16,), jnp.float32)

        def zbody(j, carry):
            acc[pl.ds(j * 16, 16)] = zero16
            return carry

        lax.fori_loop(0, ACC // 16, zbody, 0)

        ip_head = ip_v[pl.ds(0, 16)]
        ip_tail = ip_v[pl.ds(S, 16)]
        c0 = ip_head[0]
        c1 = ip_tail[0]
        k_lo = c0 // CH
        k_hi = (c1 + CH - 1) // CH

        bv = bins_v[...]
        b0 = bv[0]
        inv_w = 1.0 / jnp.full((16,), bv[1] - b0, jnp.float32)
        iota_i = lax.iota(jnp.int32, 16)
        iota17 = iota_i * 17
        for l in range(16):
            bins_x[pl.ds(l * 17, 16)] = bv

        ones_f = jnp.full((16,), 1.0, jnp.float32)

        def copy_handle(k, cbuf, sem):
            return pltpu.make_async_copy(
                coords_hbm.at[pl.ds(k * CH, CH)], cbuf, sem)

        def process_chunk(k, cbuf, carry):
            def vreg_body(v, carry):
                datas = []
                for u in range(U):
                    x = cbuf[pl.ds((v * U + u) * 16, 16)]
                    base = k * CH + (v * U + u) * 16
                    pos = base + iota_i
                    a = jnp.abs(x)
                    q = (x - b0) * inv_w
                    ch = jnp.clip(q.astype(jnp.int32), 0, 10)
                    chx = iota17 + ch
                    g1 = plsc.load_gather(bins_x, [chx])
                    g0 = plsc.load_gather(bins_x, [jnp.maximum(chx - 1,
                                                               iota17)])
                    g2 = plsc.load_gather(bins_x, [chx + 1])
                    cnt = (ch + (g1 < x).astype(jnp.int32)
                           + (g2 < x).astype(jnp.int32)
                           - ((g0 >= x) & (ch > 0)).astype(jnp.int32))
                    bin_ = jnp.clip(cnt - 1, 0, 9)
                    datas.append((pos, a, bin_, base + 15))

                p, nxt = carry
                for pos, a, bin_, g15 in datas:
                    def wcond(cc):
                        pp, nn, _ = cc
                        return (pp < S) & (nn <= g15)

                    def wbody(cc):
                        pp, nn, sv = cc
                        sv = sv + (pos >= nn).astype(jnp.int32)
                        pp = pp + 1
                        return (pp, ip_v[pl.ds(pp + 1, 16)][0], sv)

                    segv0 = jnp.full((16,), p, jnp.int32)
                    p, nxt, segv = lax.while_loop(wcond, wbody,
                                                  (p, nxt, segv0))
                    segv = jnp.where(pos >= c0, segv, S)
                    key_a = segv + 10 * SP
                    keyc = segv + bin_ * SP
                    plsc.addupdate_scatter(acc, [key_a], a)
                    plsc.addupdate_scatter(acc, [keyc], ones_f)
                return (p, nxt)

            return lax.fori_loop(0, CH // (16 * U), vreg_body, carry)

        carry0 = (jnp.int32(0), ip_head[1])

        def prologue(carry):
            copy_handle(k_lo, cbuf0, sem0).start()
            return carry

        def pair_body(j, carry):
            k0 = k_lo + 2 * j
            copy_handle(k0, cbuf0, sem0).wait()
            carry = lax.cond(
                k0 + 1 < k_hi,
                lambda c: (copy_handle(k0 + 1, cbuf1, sem1).start(), c)[1],
                lambda c: c, carry)
            carry = process_chunk(k0, cbuf0, carry)

            def half1(c):
                copy_handle(k0 + 1, cbuf1, sem1).wait()
                c = lax.cond(
                    k0 + 2 < k_hi,
                    lambda cc: (copy_handle(k0 + 2, cbuf0, sem0).start(),
                                cc)[1],
                    lambda cc: cc, c)
                return process_chunk(k0 + 1, cbuf1, c)

            return lax.cond(k0 + 1 < k_hi, half1, lambda c: c, carry)

        carry0 = lax.cond(k_lo < k_hi, prologue, lambda c: c, carry0)
        lax.fori_loop(0, (k_hi - k_lo + 1) // 2, pair_body, carry0)

        wb = [pltpu.make_async_copy(acc.at[pl.ds(c * SP, S)],
                                    out_hbm.at[pl.ds(c * n_seg + w * S, S)],
                                    sem0) for c in range(16)]
        for h in wb:
            h.start()
        for h in wb:
            h.wait()

    return sc_call


def _post_body(acc_ref, lib_ref, out_ref):
    x = acc_ref[...]
    lib = lib_ref[...][None, :, None]
    raw = x[:10]
    bc = raw / lib
    cnt = jnp.sum(raw, axis=0, keepdims=True)
    cx = jnp.log1p(jnp.sum(bc, axis=0, keepdims=True))
    asum = x[10:11]
    mean_rc = jnp.where(cnt > 0.0, asum / jnp.maximum(cnt, 1.0), 0.0) / 100000.0
    out = jnp.concatenate([
        bc,
        bc - jnp.mean(bc, axis=1, keepdims=True),
        cx,
        cx - jnp.mean(cx, axis=1, keepdims=True),
        mean_rc - jnp.mean(mean_rc, axis=1, keepdims=True),
        mean_rc,
    ], axis=0)
    out_ref[...] = out


def kernel(relative_coordinates, local_clusterxvariant_indptr, n_variants,
           n_clusters, cluster_cut_lib, bins):
    n_cuts = relative_coordinates.shape[0]
    n_seg = local_clusterxvariant_indptr.shape[0] - 1
    n_clusters_s = cluster_cut_lib.shape[0]
    n_variants_s = n_seg // n_clusters_s

    ip_pad = jnp.concatenate([
        local_clusterxvariant_indptr.astype(jnp.int32),
        jnp.full((31,), n_cuts, jnp.int32),
    ])
    bins_pad = jnp.concatenate([
        bins.astype(jnp.float32),
        jnp.full((5,), 4e9, jnp.float32),
    ])

    sc_call = _make_sc_call(n_cuts, n_seg)
    acc = sc_call(relative_coordinates, ip_pad, bins_pad)
    acc = acc.reshape(16, n_clusters_s, n_variants_s)

    out_t = pl.pallas_call(
        _post_body,
        out_shape=jax.ShapeDtypeStruct((24, n_clusters_s, n_variants_s),
                                       jnp.float32),
    )(acc, cluster_cut_lib)
    return jnp.transpose(out_t, (1, 2, 0))


# speedup vs baseline: 1.2871x; 1.2871x over previous
import functools

import jax
import jax.numpy as jnp
from jax import lax
from jax.experimental import pallas as pl
from jax.experimental.pallas import tpu as pltpu
from jax.experimental.pallas import tpu_sc as plsc

NC = 2
NS = 16
NW = NC * NS
L = 16
CH = 2048
U = 8


def _make_sc_call(n_cuts, n_seg):
    S = n_seg // NW
    SP = S + 8
    NCHAN = 20
    ACC = NCHAN * SP
    IPW = S + 32
    mesh = plsc.VectorSubcoreMesh(core_axis_name="c", subcore_axis_name="s")

    @functools.partial(
        pl.kernel,
        mesh=mesh,
        compiler_params=pltpu.CompilerParams(needs_layout_passes=False),
        out_type=jax.ShapeDtypeStruct((NCHAN * n_seg,), jnp.float32),
        scratch_types=[
            pltpu.VMEM((IPW,), jnp.int32),
            pltpu.VMEM((16,), jnp.float32),
            pltpu.VMEM((CH,), jnp.float32),
            pltpu.VMEM((CH,), jnp.float32),
            pltpu.VMEM((ACC,), jnp.float32),
            pltpu.SemaphoreType.DMA,
            pltpu.SemaphoreType.DMA,
        ],
    )
    def sc_call(coords_hbm, ip_hbm, bins_hbm, out_hbm,
                ip_v, bins_v, cbuf0, cbuf1, acc, sem0, sem1):
        cid = lax.axis_index("c")
        sid = lax.axis_index("s")
        w = sid * NC + cid
        pltpu.sync_copy(ip_hbm.at[pl.ds(w * S, IPW)], ip_v)
        pltpu.sync_copy(bins_hbm, bins_v)

        zero16 = jnp.zeros((16,), jnp.float32)

        def zbody(j, carry):
            acc[pl.ds(j * 16, 16)] = zero16
            return carry

        lax.fori_loop(0, ACC // 16, zbody, 0)

        ip_head = ip_v[pl.ds(0, 16)]
        ip_tail = ip_v[pl.ds(S, 16)]
        c0 = ip_head[0]
        c1 = ip_tail[0]
        k_lo = c0 // CH
        k_hi = (c1 + CH - 1) // CH

        bv = bins_v[...]
        b0 = bv[0]
        inv_w = 1.0 / jnp.full((16,), bv[1] - b0, jnp.float32)
        iota_i = lax.iota(jnp.int32, 16)

        ones_f = jnp.full((16,), 1.0, jnp.float32)

        def copy_handle(k, cbuf, sem):
            return pltpu.make_async_copy(
                coords_hbm.at[pl.ds(k * CH, CH)], cbuf, sem)

        def process_chunk(k, cbuf, carry):
            def vreg_body(v, carry):
                datas = []
                for u in range(U):
                    x = cbuf[pl.ds((v * U + u) * 16, 16)]
                    base = k * CH + (v * U + u) * 16
                    pos = base + iota_i
                    a = jnp.abs(x)
                    q = (x - b0) * inv_w
                    ch = jnp.clip(q.astype(jnp.int32), 0, 10)
                    g1 = plsc.load_gather(bins_v, [ch])
                    g0 = plsc.load_gather(bins_v, [jnp.maximum(ch - 1, 0)])
                    g2 = plsc.load_gather(bins_v, [ch + 1])
                    cnt = (ch + (g1 < x).astype(jnp.int32)
                           + (g2 < x).astype(jnp.int32)
                           - ((g0 >= x) & (ch > 0)).astype(jnp.int32))
                    bin_ = jnp.clip(cnt - 1, 0, 9)
                    datas.append((pos, a, bin_, base + 15))

                p, nxt = carry
                for pos, a, bin_, g15 in datas:
                    def wcond(cc):
                        pp, nn, _ = cc
                        return (pp < S) & (nn <= g15)

                    def wbody(cc):
                        pp, nn, sv = cc
                        sv = sv + (pos >= nn).astype(jnp.int32)
                        pp = pp + 1
                        return (pp, ip_v[pl.ds(pp + 1, 16)][0], sv)

                    segv0 = jnp.full((16,), p, jnp.int32)
                    p, nxt, segv = lax.while_loop(wcond, wbody,
                                                  (p, nxt, segv0))
                    segv = jnp.where(pos >= c0, segv, S)
                    keyc = segv + bin_ * SP
                    plsc.addupdate_scatter(acc, [keyc], ones_f)
                    plsc.addupdate_scatter(acc, [keyc + 10 * SP], a)
                return (p, nxt)

            return lax.fori_loop(0, CH // (16 * U), vreg_body, carry)

        carry0 = (jnp.int32(0), ip_head[1])

        def prologue(carry):
            copy_handle(k_lo, cbuf0, sem0).start()
            return carry

        def pair_body(j, carry):
            k0 = k_lo + 2 * j
            copy_handle(k0, cbuf0, sem0).wait()
            carry = lax.cond(
                k0 + 1 < k_hi,
                lambda c: (copy_handle(k0 + 1, cbuf1, sem1).start(), c)[1],
                lambda c: c, carry)
            carry = process_chunk(k0, cbuf0, carry)

            def half1(c):
                copy_handle(k0 + 1, cbuf1, sem1).wait()
                c = lax.cond(
                    k0 + 2 < k_hi,
                    lambda cc: (copy_handle(k0 + 2, cbuf0, sem0).start(),
                                cc)[1],
                    lambda cc: cc, c)
                return process_chunk(k0 + 1, cbuf1, c)

            return lax.cond(k0 + 1 < k_hi, half1, lambda c: c, carry)

        carry0 = lax.cond(k_lo < k_hi, prologue, lambda c: c, carry0)
        lax.fori_loop(0, (k_hi - k_lo + 1) // 2, pair_body, carry0)

        wb = [pltpu.make_async_copy(acc.at[pl.ds(c * SP, S)],
                                    out_hbm.at[pl.ds(c * n_seg + w * S, S)],
                                    sem0) for c in range(NCHAN)]
        for h in wb:
            h.start()
        for h in wb:
            h.wait()

    return sc_call


def _post_body(acc_ref, lib_ref, out_ref):
    x = acc_ref[...]
    lib = lib_ref[...][None, :, None]
    raw = x[:10]
    bc = raw / lib
    cnt = jnp.sum(raw, axis=0, keepdims=True)
    cx = jnp.log1p(jnp.sum(bc, axis=0, keepdims=True))
    asum = jnp.sum(x[10:20], axis=0, keepdims=True)
    mean_rc = jnp.where(cnt > 0.0, asum / jnp.maximum(cnt, 1.0), 0.0) / 100000.0
    out = jnp.concatenate([
        bc,
        bc - jnp.mean(bc, axis=1, keepdims=True),
        cx,
        cx - jnp.mean(cx, axis=1, keepdims=True),
        mean_rc - jnp.mean(mean_rc, axis=1, keepdims=True),
        mean_rc,
    ], axis=0)
    out_ref[...] = out


def kernel(relative_coordinates, local_clusterxvariant_indptr, n_variants,
           n_clusters, cluster_cut_lib, bins):
    n_cuts = relative_coordinates.shape[0]
    n_seg = local_clusterxvariant_indptr.shape[0] - 1
    n_clusters_s = cluster_cut_lib.shape[0]
    n_variants_s = n_seg // n_clusters_s

    ip_pad = jnp.concatenate([
        local_clusterxvariant_indptr.astype(jnp.int32),
        jnp.full((31,), n_cuts, jnp.int32),
    ])
    bins_pad = jnp.concatenate([
        bins.astype(jnp.float32),
        jnp.full((5,), 4e9, jnp.float32),
    ])

    sc_call = _make_sc_call(n_cuts, n_seg)
    acc = sc_call(relative_coordinates, ip_pad, bins_pad)
    acc = acc.reshape(20, n_clusters_s, n_variants_s)

    out_t = pl.pallas_call(
        _post_body,
        out_shape=jax.ShapeDtypeStruct((24, n_clusters_s, n_variants_s),
                                       jnp.float32),
    )(acc, cluster_cut_lib)
    return jnp.transpose(out_t, (1, 2, 0))


# speedup vs baseline: 1.4459x; 1.1233x over previous
import functools

import jax
import jax.numpy as jnp
from jax import lax
from jax.experimental import pallas as pl
from jax.experimental.pallas import tpu as pltpu
from jax.experimental.pallas import tpu_sc as plsc

NC = 2
NS = 16
NW = NC * NS
L = 16
CH = 2048
U = 8


def _make_sc_call(n_cuts, n_seg):
    S = n_seg // NW
    SP = S + 8
    NCHAN = 20
    ACC = NCHAN * SP
    IPW = S + 32
    mesh = plsc.VectorSubcoreMesh(core_axis_name="c", subcore_axis_name="s")

    @functools.partial(
        pl.kernel,
        mesh=mesh,
        compiler_params=pltpu.CompilerParams(needs_layout_passes=False),
        out_type=jax.ShapeDtypeStruct((NCHAN * n_seg,), jnp.float32),
        scratch_types=[
            pltpu.VMEM((IPW,), jnp.int32),
            pltpu.VMEM((16,), jnp.float32),
            pltpu.VMEM((CH,), jnp.float32),
            pltpu.VMEM((CH,), jnp.float32),
            pltpu.VMEM((ACC,), jnp.float32),
            pltpu.SemaphoreType.DMA,
            pltpu.SemaphoreType.DMA,
        ],
    )
    def sc_call(coords_hbm, ip_hbm, bins_hbm, out_hbm,
                ip_v, bins_v, cbuf0, cbuf1, acc, sem0, sem1):
        cid = lax.axis_index("c")
        sid = lax.axis_index("s")
        w = sid * NC + cid
        pltpu.sync_copy(ip_hbm.at[pl.ds(w * S, IPW)], ip_v)
        pltpu.sync_copy(bins_hbm, bins_v)

        zero16 = jnp.zeros((16,), jnp.float32)

        def zbody(j, carry):
            acc[pl.ds(j * 16, 16)] = zero16
            return carry

        lax.fori_loop(0, ACC // 16, zbody, 0)

        ip_head = ip_v[pl.ds(0, 16)]
        ip_tail = ip_v[pl.ds(S, 16)]
        c0 = ip_head[0]
        c1 = ip_tail[0]
        k_lo = c0 // CH
        k_hi = (c1 + CH - 1) // CH

        bv = bins_v[...]
        b0 = bv[0]
        inv_w = 1.0 / jnp.full((16,), bv[1] - b0, jnp.float32)
        iota_i = lax.iota(jnp.int32, 16)

        ones_f = jnp.full((16,), 1.0, jnp.float32)

        def copy_handle(k, cbuf, sem):
            return pltpu.make_async_copy(
                coords_hbm.at[pl.ds(k * CH, CH)], cbuf, sem)

        def process_chunk(k, cbuf, carry):
            def vreg_body(v, carry):
                datas = []
                for u in range(U):
                    x = cbuf[pl.ds((v * U + u) * 16, 16)]
                    base = k * CH + (v * U + u) * 16
                    pos = base + iota_i
                    a = jnp.abs(x)
                    q = (x - b0) * inv_w
                    ch = jnp.clip(q.astype(jnp.int32), 0, 10)
                    g1 = plsc.load_gather(bins_v, [ch])
                    g0 = plsc.load_gather(bins_v, [jnp.maximum(ch - 1, 0)])
                    g2 = plsc.load_gather(bins_v, [ch + 1])
                    cnt = (ch + (g1 < x).astype(jnp.int32)
                           + (g2 < x).astype(jnp.int32)
                           - ((g0 >= x) & (ch > 0)).astype(jnp.int32))
                    bin_ = jnp.clip(cnt - 1, 0, 9)
                    datas.append((pos, a, bin_, base + 15))

                p, nxt = carry
                g_last = k * CH + (v * U + U) * 16 - 1

                def wcond(cc):
                    pp, nn = cc[0], cc[1]
                    return (pp < S) & (nn <= g_last)

                def wbody(cc):
                    pp, nn = cc[0], cc[1]
                    svs = tuple(
                        sv + (d[0] >= nn).astype(jnp.int32)
                        for sv, d in zip(cc[2:], datas))
                    pp = pp + 1
                    return (pp, ip_v[pl.ds(pp + 1, 16)][0]) + svs

                init = (p, nxt) + tuple(
                    jnp.full((16,), p, jnp.int32) for _ in range(U))
                res = lax.while_loop(wcond, wbody, init)
                p, nxt = res[0], res[1]
                for segv, (pos, a, bin_, g15) in zip(res[2:], datas):
                    segv = jnp.where(pos >= c0, segv, S)
                    keyc = segv + bin_ * SP
                    plsc.addupdate_scatter(acc, [keyc], ones_f)
                    plsc.addupdate_scatter(acc, [keyc + 10 * SP], a)
                return (p, nxt)

            return lax.fori_loop(0, CH // (16 * U), vreg_body, carry)

        carry0 = (jnp.int32(0), ip_head[1])

        def prologue(carry):
            copy_handle(k_lo, cbuf0, sem0).start()
            return carry

        def pair_body(j, carry):
            k0 = k_lo + 2 * j
            copy_handle(k0, cbuf0, sem0).wait()
            carry = lax.cond(
                k0 + 1 < k_hi,
                lambda c: (copy_handle(k0 + 1, cbuf1, sem1).start(), c)[1],
                lambda c: c, carry)
            carry = process_chunk(k0, cbuf0, carry)

            def half1(c):
                copy_handle(k0 + 1, cbuf1, sem1).wait()
                c = lax.cond(
                    k0 + 2 < k_hi,
                    lambda cc: (copy_handle(k0 + 2, cbuf0, sem0).start(),
                                cc)[1],
                    lambda cc: cc, c)
                return process_chunk(k0 + 1, cbuf1, c)

            return lax.cond(k0 + 1 < k_hi, half1, lambda c: c, carry)

        carry0 = lax.cond(k_lo < k_hi, prologue, lambda c: c, carry0)
        lax.fori_loop(0, (k_hi - k_lo + 1) // 2, pair_body, carry0)

        wb = [pltpu.make_async_copy(acc.at[pl.ds(c * SP, S)],
                                    out_hbm.at[pl.ds(c * n_seg + w * S, S)],
                                    sem0) for c in range(NCHAN)]
        for h in wb:
            h.start()
        for h in wb:
            h.wait()

    return sc_call


def _post_body(acc_ref, lib_ref, out_ref):
    x = acc_ref[...]
    lib = lib_ref[...][None, :, None]
    raw = x[:10]
    bc = raw / lib
    cnt = jnp.sum(raw, axis=0, keepdims=True)
    cx = jnp.log1p(jnp.sum(bc, axis=0, keepdims=True))
    asum = jnp.sum(x[10:20], axis=0, keepdims=True)
    mean_rc = jnp.where(cnt > 0.0, asum / jnp.maximum(cnt, 1.0), 0.0) / 100000.0
    out = jnp.concatenate([
        bc,
        bc - jnp.mean(bc, axis=1, keepdims=True),
        cx,
        cx - jnp.mean(cx, axis=1, keepdims=True),
        mean_rc - jnp.mean(mean_rc, axis=1, keepdims=True),
        mean_rc,
    ], axis=0)
    out_ref[...] = out


def kernel(relative_coordinates, local_clusterxvariant_indptr, n_variants,
           n_clusters, cluster_cut_lib, bins):
    n_cuts = relative_coordinates.shape[0]
    n_seg = local_clusterxvariant_indptr.shape[0] - 1
    n_clusters_s = cluster_cut_lib.shape[0]
    n_variants_s = n_seg // n_clusters_s

    ip_pad = jnp.concatenate([
        local_clusterxvariant_indptr.astype(jnp.int32),
        jnp.full((31,), n_cuts, jnp.int32),
    ])
    bins_pad = jnp.concatenate([
        bins.astype(jnp.float32),
        jnp.full((5,), 4e9, jnp.float32),
    ])

    sc_call = _make_sc_call(n_cuts, n_seg)
    acc = sc_call(relative_coordinates, ip_pad, bins_pad)
    acc = acc.reshape(20, n_clusters_s, n_variants_s)

    out_t = pl.pallas_call(
        _post_body,
        out_shape=jax.ShapeDtypeStruct((24, n_clusters_s, n_variants_s),
                                       jnp.float32),
    )(acc, cluster_cut_lib)
    return jnp.transpose(out_t, (1, 2, 0))


# speedup vs baseline: 1.5679x; 1.0844x over previous
import functools

import jax
import jax.numpy as jnp
from jax import lax
from jax.experimental import pallas as pl
from jax.experimental.pallas import tpu as pltpu
from jax.experimental.pallas import tpu_sc as plsc

NC = 2
NS = 16
NW = NC * NS
L = 16
CH = 2048
U = 8


def _make_sc_call(n_cuts, n_seg):
    S = n_seg // NW
    SP = S + 8
    NCHAN = 20
    ACC = NCHAN * SP
    IPW = S + 32
    mesh = plsc.VectorSubcoreMesh(core_axis_name="c", subcore_axis_name="s")

    @functools.partial(
        pl.kernel,
        mesh=mesh,
        compiler_params=pltpu.CompilerParams(needs_layout_passes=False),
        out_type=jax.ShapeDtypeStruct((NCHAN * n_seg,), jnp.float32),
        scratch_types=[
            pltpu.VMEM((IPW,), jnp.int32),
            pltpu.VMEM((16,), jnp.float32),
            pltpu.VMEM((CH,), jnp.float32),
            pltpu.VMEM((CH,), jnp.float32),
            pltpu.VMEM((ACC,), jnp.float32),
            pltpu.SemaphoreType.DMA,
            pltpu.SemaphoreType.DMA,
        ],
    )
    def sc_call(coords_hbm, ip_hbm, bins_hbm, out_hbm,
                ip_v, bins_v, cbuf0, cbuf1, acc, sem0, sem1):
        cid = lax.axis_index("c")
        sid = lax.axis_index("s")
        w = sid * NC + cid
        pltpu.sync_copy(ip_hbm.at[pl.ds(w * S, IPW)], ip_v)
        pltpu.sync_copy(bins_hbm, bins_v)

        zero16 = jnp.zeros((16,), jnp.float32)

        def zbody(j, carry):
            acc[pl.ds(j * 16, 16)] = zero16
            return carry

        lax.fori_loop(0, ACC // 16, zbody, 0)

        ip_head = ip_v[pl.ds(0, 16)]
        ip_tail = ip_v[pl.ds(S, 16)]
        c0 = ip_head[0]
        c1 = ip_tail[0]
        k_lo = c0 // CH
        k_hi = (c1 + CH - 1) // CH

        bv = bins_v[...]
        b0 = bv[0]
        inv_w = 1.0 / jnp.full((16,), bv[1] - b0, jnp.float32)
        iota_i = lax.iota(jnp.int32, 16)

        ones_f = jnp.full((16,), 1.0, jnp.float32)

        def copy_handle(k, cbuf, sem):
            return pltpu.make_async_copy(
                coords_hbm.at[pl.ds(k * CH, CH)], cbuf, sem)

        def process_chunk(k, cbuf, carry):
            def vreg_body(v, carry):
                datas = []
                for u in range(U):
                    x = cbuf[pl.ds((v * U + u) * 16, 16)]
                    base = k * CH + (v * U + u) * 16
                    pos = base + iota_i
                    a = jnp.abs(x)
                    q = (x - b0) * inv_w
                    ch = jnp.clip(q.astype(jnp.int32), 0, 10)
                    g1 = plsc.load_gather(bins_v, [ch])
                    g2 = plsc.load_gather(bins_v, [ch + 1])
                    cnt = (ch + (g1 < x).astype(jnp.int32)
                           + (g2 < x).astype(jnp.int32))
                    bin_ = jnp.clip(cnt - 1, 0, 9)
                    datas.append((pos, a, bin_, base + 15))

                p, nxt = carry
                g_last = k * CH + (v * U + U) * 16 - 1

                def wcond(cc):
                    pp, nn = cc[0], cc[1]
                    return (pp < S) & (nn <= g_last)

                def wbody(cc):
                    pp, nn = cc[0], cc[1]
                    svs = tuple(
                        sv + (d[0] >= nn).astype(jnp.int32)
                        for sv, d in zip(cc[2:], datas))
                    pp = pp + 1
                    return (pp, ip_v[pl.ds(pp + 1, 16)][0]) + svs

                init = (p, nxt) + tuple(
                    jnp.full((16,), p, jnp.int32) for _ in range(U))
                res = lax.while_loop(wcond, wbody, init)
                p, nxt = res[0], res[1]
                for segv, (pos, a, bin_, g15) in zip(res[2:], datas):
                    segv = jnp.where(pos >= c0, segv, S)
                    keyc = segv + bin_ * SP
                    plsc.addupdate_scatter(acc, [keyc], ones_f)
                    plsc.addupdate_scatter(acc, [keyc + 10 * SP], a)
                return (p, nxt)

            return lax.fori_loop(0, CH // (16 * U), vreg_body, carry)

        carry0 = (jnp.int32(0), ip_head[1])

        def prologue(carry):
            copy_handle(k_lo, cbuf0, sem0).start()
            return carry

        def pair_body(j, carry):
            k0 = k_lo + 2 * j
            copy_handle(k0, cbuf0, sem0).wait()
            carry = lax.cond(
                k0 + 1 < k_hi,
                lambda c: (copy_handle(k0 + 1, cbuf1, sem1).start(), c)[1],
                lambda c: c, carry)
            carry = process_chunk(k0, cbuf0, carry)

            def half1(c):
                copy_handle(k0 + 1, cbuf1, sem1).wait()
                c = lax.cond(
                    k0 + 2 < k_hi,
                    lambda cc: (copy_handle(k0 + 2, cbuf0, sem0).start(),
                                cc)[1],
                    lambda cc: cc, c)
                return process_chunk(k0 + 1, cbuf1, c)

            return lax.cond(k0 + 1 < k_hi, half1, lambda c: c, carry)

        carry0 = lax.cond(k_lo < k_hi, prologue, lambda c: c, carry0)
        lax.fori_loop(0, (k_hi - k_lo + 1) // 2, pair_body, carry0)

        wb = [pltpu.make_async_copy(acc.at[pl.ds(c * SP, S)],
                                    out_hbm.at[pl.ds(c * n_seg + w * S, S)],
                                    sem0) for c in range(NCHAN)]
        for h in wb:
            h.start()
        for h in wb:
            h.wait()

    return sc_call


def _post_body(acc_ref, lib_ref, out_ref):
    x = acc_ref[...]
    lib = lib_ref[...][None, :, None]
    raw = x[:10]
    bc = raw / lib
    cnt = jnp.sum(raw, axis=0, keepdims=True)
    cx = jnp.log1p(jnp.sum(bc, axis=0, keepdims=True))
    asum = jnp.sum(x[10:20], axis=0, keepdims=True)
    mean_rc = jnp.where(cnt > 0.0, asum / jnp.maximum(cnt, 1.0), 0.0) / 100000.0
    out = jnp.concatenate([
        bc,
        bc - jnp.mean(bc, axis=1, keepdims=True),
        cx,
        cx - jnp.mean(cx, axis=1, keepdims=True),
        mean_rc - jnp.mean(mean_rc, axis=1, keepdims=True),
        mean_rc,
    ], axis=0)
    out_ref[...] = out


def kernel(relative_coordinates, local_clusterxvariant_indptr, n_variants,
           n_clusters, cluster_cut_lib, bins):
    n_cuts = relative_coordinates.shape[0]
    n_seg = local_clusterxvariant_indptr.shape[0] - 1
    n_clusters_s = cluster_cut_lib.shape[0]
    n_variants_s = n_seg // n_clusters_s

    ip_pad = jnp.concatenate([
        local_clusterxvariant_indptr.astype(jnp.int32),
        jnp.full((31,), n_cuts, jnp.int32),
    ])
    bins_pad = jnp.concatenate([
        bins.astype(jnp.float32),
        jnp.full((5,), 4e9, jnp.float32),
    ])

    sc_call = _make_sc_call(n_cuts, n_seg)
    acc = sc_call(relative_coordinates, ip_pad, bins_pad)
    acc = acc.reshape(20, n_clusters_s, n_variants_s)

    out_t = pl.pallas_call(
        _post_body,
        out_shape=jax.ShapeDtypeStruct((24, n_clusters_s, n_variants_s),
                                       jnp.float32),
    )(acc, cluster_cut_lib)
    return jnp.transpose(out_t, (1, 2, 0))
